# baseline (speedup 1.0000x reference)
import jax
import jax.numpy as jnp
from jax import lax
from jax.experimental import pallas as pl
from jax.experimental.pallas import tpu as pltpu
from jax.experimental.pallas import tpu_sc as plsc

VOCAB = 1000000
EMB = 64
OUT = 64
B = 4096
L = 200

NC = 2
NS = 16
NW = NC * NS
E_PER_W = B // NW
IDX_PER_W = E_PER_W * L
LPAD = 224

SLOT = 512000
CROWS = SLOT
PBLK = 2048
NBLK = SLOT // PBLK

CHUNKS = ((0, 120), (120, 80))
UNROLL = 8
NSLICE = (L + 15) // 16

MASK_HI = -65536
ROUND = 0x8000


def _widen_body(t0, t1, w_ref, o_ref):
    dn = (((0,), (0,)), ((), ()))
    p0 = lax.dot_general(t0[...], w_ref[...], dn,
                         preferred_element_type=jnp.float32)
    p1 = lax.dot_general(t1[...], w_ref[...], dn,
                         preferred_element_type=jnp.float32)
    o_ref[...] = jnp.concatenate([p0, p1], axis=1)


def _widen(table_t, W):
    in_specs = [
        pl.BlockSpec((EMB, PBLK),
                     (lambda s: lambda i: (0, jnp.minimum(i + s * NBLK,
                                                          VOCAB // PBLK)))(s))
        for s in range(2)
    ] + [pl.BlockSpec((EMB, OUT), lambda i: (0, 0))]
    return pl.pallas_call(
        _widen_body,
        grid=(NBLK,),
        in_specs=in_specs,
        out_specs=pl.BlockSpec((PBLK, 2 * EMB), lambda i: (i, 0)),
        out_shape=jax.ShapeDtypeStruct((CROWS, 2 * EMB), jnp.float32),
    )(table_t, table_t, W)


def _pool_body(x_hbm, t2_hbm, b_hbm, out_hbm, idx_v, pidx0_v, pidx1_v,
               rows_v, pooled_v, b_v, sem0, sem1):
    c = lax.axis_index("c")
    s = lax.axis_index("s")
    wid = s * NC + c
    base_e = wid * E_PER_W
    sems = (sem0, sem1)
    pidxs = (pidx0_v, pidx1_v)
    last = jnp.int32(E_PER_W - 1)
    lanes = lax.iota(jnp.int32, 16)
    tail_valid = lanes < (L - (NSLICE - 1) * 16)

    pltpu.sync_copy(x_hbm.at[pl.ds(base_e * L, IDX_PER_W)],
                    idx_v.at[pl.ds(0, IDX_PER_W)])
    pltpu.sync_copy(b_hbm, b_v)

    def prep(e, buf):
        ebase = e * L
        ptr = jnp.zeros((16,), jnp.int32)
        n1 = jnp.int32(0)
        dst = pidxs[buf]
        for want in range(2):
            for u in range(NSLICE):
                v = idx_v[pl.ds(ebase + u * 16, 16)]
                hi = v >= SLOT
                h = jnp.where(hi, v - SLOT, v)
                m = hi if want else jnp.logical_not(hi)
                if u == NSLICE - 1:
                    m = jnp.logical_and(m, tail_valid)
                pos = ptr + plsc.cumsum(m.astype(jnp.int32)) - 1
                plsc.store_scatter(dst, [pos], h, mask=m)
                ptr = ptr + plsc.all_reduce_population_count(m)
            if want == 0:
                n1 = ptr[0]
        return (n1,)

    def copies(buf):
        return [
            pltpu.make_async_copy(
                t2_hbm.at[pidxs[buf].at[pl.ds(off, n)]],
                rows_v.at[buf].at[pl.ds(off, n)],
                sems[buf],
            )
            for off, n in CHUNKS
        ]

    def fire(buf):
        for cp in copies(buf):
            cp.start()

    def wait(buf):
        for cp in copies(buf):
            cp.wait()

    def reduce_into(e, buf, cuts):
        (n1,) = cuts

        def red(i, accs):
            r = i * UNROLL
            out = list(accs)
            for rr in range(UNROLL):
                rg = r + rr
                off = jnp.where(rg < n1, 0, EMB)
                for j in range(4):
                    out[j] = out[j] + rows_v[buf, rg,
                                             pl.ds(off + j * 16, 16)]
            return tuple(out)

        z = jnp.zeros((16,), jnp.float32)
        acc = lax.fori_loop(0, L // UNROLL, red, (z,) * 4, unroll=1)
        scale = jnp.float32(1.0 / L)
        for j in range(4):
            pooled_v[e, pl.ds(j * 16, 16)] = (acc[j] * scale
                                              + b_v[pl.ds(j * 16, 16)])

    cuts0 = prep(jnp.int32(0), 0)
    fire(0)
    cuts1 = prep(jnp.int32(1), 1)
    fire(1)

    def pair(i, carry):
        cuts0 = carry[0:1]
        cuts1 = carry[1:2]
        e0 = 2 * i
        wait(0)
        reduce_into(e0, 0, cuts0)
        cuts0n = prep(jnp.minimum(e0 + 2, last), 0)
        fire(0)
        wait(1)
        reduce_into(e0 + 1, 1, cuts1)
        cuts1n = prep(jnp.minimum(e0 + 3, last), 1)
        fire(1)
        return cuts0n + cuts1n

    lax.fori_loop(0, E_PER_W // 2, pair, cuts0 + cuts1)
    wait(0)
    wait(1)
    pltpu.sync_copy(pooled_v, out_hbm.at[pl.ds(base_e, E_PER_W)])


@jax.jit
def _pool(x_flat, table2, b):
    mesh = plsc.VectorSubcoreMesh(core_axis_name="c", subcore_axis_name="s")
    return pl.kernel(
        _pool_body,
        out_type=jax.ShapeDtypeStruct((B, EMB), jnp.float32),
        mesh=mesh,
        scratch_types=[
            pltpu.VMEM((IDX_PER_W + 16,), jnp.int32),
            pltpu.VMEM((LPAD,), jnp.int32),
            pltpu.VMEM((LPAD,), jnp.int32),
            pltpu.VMEM((2, L, 2 * EMB), jnp.float32),
            pltpu.VMEM((E_PER_W, EMB), jnp.float32),
            pltpu.VMEM((EMB,), jnp.float32),
            pltpu.SemaphoreType.DMA,
            pltpu.SemaphoreType.DMA,
        ],
        compiler_params=pltpu.CompilerParams(needs_layout_passes=False),
    )(x_flat, table2, b)


def kernel(x, table, W, b):
    x_flat = x.reshape(-1).astype(jnp.int32)
    table2 = _widen(table.T, W)
    return _pool(x_flat, table2, b)


# speedup vs baseline: 1.1533x; 1.1533x over previous
import jax
import jax.numpy as jnp
from jax import lax
from jax.experimental import pallas as pl
from jax.experimental.pallas import tpu as pltpu
from jax.experimental.pallas import tpu_sc as plsc

VOCAB = 1000000
EMB = 64
OUT = 64
B = 4096
L = 200

NC = 2
NS = 16
NW = NC * NS
E_PER_W = B // NW
IDX_PER_W = E_PER_W * L
LPAD = 224

SLOT = 512000
CROWS = SLOT
PBLK = 4096
NBLK = SLOT // PBLK

CHUNKS = ((0, 120), (120, 80))
UNROLL = 8
NSLICE = (L + 15) // 16

MASK_HI = -65536
ROUND = 0x8000


def _widen_body(t0, t1, w_ref, o_ref):
    dn = (((0,), (0,)), ((), ()))
    p0 = lax.dot_general(t0[...], w_ref[...], dn,
                         preferred_element_type=jnp.float32)
    p1 = lax.dot_general(t1[...], w_ref[...], dn,
                         preferred_element_type=jnp.float32)
    o_ref[...] = jnp.concatenate([p0, p1], axis=1)


def _widen(table_t, W):
    in_specs = [
        pl.BlockSpec((EMB, PBLK),
                     (lambda s: lambda i: (0, jnp.minimum(i + s * NBLK,
                                                          VOCAB // PBLK)))(s))
        for s in range(2)
    ] + [pl.BlockSpec((EMB, OUT), lambda i: (0, 0))]
    return pl.pallas_call(
        _widen_body,
        grid=(NBLK,),
        in_specs=in_specs,
        out_specs=pl.BlockSpec((PBLK, 2 * EMB), lambda i: (i, 0)),
        out_shape=jax.ShapeDtypeStruct((CROWS, 2 * EMB), jnp.float32),
    )(table_t, table_t, W)


def _pool_body(x_hbm, t2_hbm, b_hbm, out_hbm, idx_v, pidx0_v, pidx1_v,
               rows_v, pooled_v, b_v, sem0, sem1):
    c = lax.axis_index("c")
    s = lax.axis_index("s")
    wid = s * NC + c
    base_e = wid * E_PER_W
    sems = (sem0, sem1)
    pidxs = (pidx0_v, pidx1_v)
    last = jnp.int32(E_PER_W - 1)
    lanes = lax.iota(jnp.int32, 16)
    tail_valid = lanes < (L - (NSLICE - 1) * 16)

    pltpu.sync_copy(x_hbm.at[pl.ds(base_e * L, IDX_PER_W)],
                    idx_v.at[pl.ds(0, IDX_PER_W)])
    pltpu.sync_copy(b_hbm, b_v)

    def prep(e, buf):
        ebase = e * L
        ptr = jnp.zeros((16,), jnp.int32)
        n1 = jnp.int32(0)
        dst = pidxs[buf]
        for want in range(2):
            for u in range(NSLICE):
                v = idx_v[pl.ds(ebase + u * 16, 16)]
                hi = v >= SLOT
                h = jnp.where(hi, v - SLOT, v)
                m = hi if want else jnp.logical_not(hi)
                if u == NSLICE - 1:
                    m = jnp.logical_and(m, tail_valid)
                pos = ptr + plsc.cumsum(m.astype(jnp.int32)) - 1
                plsc.store_scatter(dst, [pos], h, mask=m)
                ptr = ptr + plsc.all_reduce_population_count(m)
            if want == 0:
                n1 = ptr[0]
        return (n1,)

    def copies(buf):
        return [
            pltpu.make_async_copy(
                t2_hbm.at[pidxs[buf].at[pl.ds(off, n)]],
                rows_v.at[buf].at[pl.ds(off, n)],
                sems[buf],
            )
            for off, n in CHUNKS
        ]

    def fire(buf):
        for cp in copies(buf):
            cp.start()

    def wait(buf):
        for cp in copies(buf):
            cp.wait()

    def reduce_into(e, buf, cuts):
        (n1,) = cuts

        def red(i, accs):
            r = i * UNROLL
            out = list(accs)
            for rr in range(UNROLL):
                rg = r + rr
                off = jnp.where(rg < n1, 0, EMB)
                for j in range(4):
                    out[j] = out[j] + rows_v[buf, rg,
                                             pl.ds(off + j * 16, 16)]
            return tuple(out)

        z = jnp.zeros((16,), jnp.float32)
        acc = lax.fori_loop(0, L // UNROLL, red, (z,) * 4, unroll=1)
        scale = jnp.float32(1.0 / L)
        for j in range(4):
            pooled_v[e, pl.ds(j * 16, 16)] = (acc[j] * scale
                                              + b_v[pl.ds(j * 16, 16)])

    cuts0 = prep(jnp.int32(0), 0)
    fire(0)
    cuts1 = prep(jnp.int32(1), 1)
    fire(1)

    def pair(i, carry):
        cuts0 = carry[0:1]
        cuts1 = carry[1:2]
        e0 = 2 * i
        wait(0)
        reduce_into(e0, 0, cuts0)
        cuts0n = prep(jnp.minimum(e0 + 2, last), 0)
        fire(0)
        wait(1)
        reduce_into(e0 + 1, 1, cuts1)
        cuts1n = prep(jnp.minimum(e0 + 3, last), 1)
        fire(1)
        return cuts0n + cuts1n

    lax.fori_loop(0, E_PER_W // 2, pair, cuts0 + cuts1)
    wait(0)
    wait(1)
    pltpu.sync_copy(pooled_v, out_hbm.at[pl.ds(base_e, E_PER_W)])


@jax.jit
def _pool(x_flat, table2, b):
    mesh = plsc.VectorSubcoreMesh(core_axis_name="c", subcore_axis_name="s")
    return pl.kernel(
        _pool_body,
        out_type=jax.ShapeDtypeStruct((B, EMB), jnp.float32),
        mesh=mesh,
        scratch_types=[
            pltpu.VMEM((IDX_PER_W + 16,), jnp.int32),
            pltpu.VMEM((LPAD,), jnp.int32),
            pltpu.VMEM((LPAD,), jnp.int32),
            pltpu.VMEM((2, L, 2 * EMB), jnp.float32),
            pltpu.VMEM((E_PER_W, EMB), jnp.float32),
            pltpu.VMEM((EMB,), jnp.float32),
            pltpu.SemaphoreType.DMA,
            pltpu.SemaphoreType.DMA,
        ],
        compiler_params=pltpu.CompilerParams(needs_layout_passes=False),
    )(x_flat, table2, b)


def kernel(x, table, W, b):
    x_flat = x.reshape(-1).astype(jnp.int32)
    table2 = _widen(table.T, W)
    return _pool(x_flat, table2, b)
